# baseline (speedup 1.0000x reference)
import functools

import jax
import jax.numpy as jnp
from jax import lax
from jax.experimental import pallas as pl
from jax.experimental.pallas import tpu as pltpu
from jax.experimental.pallas import tpu_sc as plsc

L = 16
NC = 2
NS = 16
NW = NC * NS
ROWS = 16384
COLS = 200
RPW = ROWS // NW
GROUPS = RPW // L
CHUNK = 256


def _body(idx_hbm, es_hbm, out_hbm, buf, es_v, out_v):
    wid = lax.axis_index("s") * NC + lax.axis_index("c")

    pltpu.sync_copy(es_hbm, es_v.at[pl.ds(0, 4)])

    ev = es_v[pl.ds(0, L)]
    e_splat = [jnp.full((L,), ev[k], jnp.float32) for k in range(4)]

    lane_rows = lax.iota(jnp.int32, L)

    for c in range(RPW // CHUNK):
        pltpu.sync_copy(
            idx_hbm.at[pl.ds(wid * RPW + c * CHUNK, CHUNK), :], buf
        )
        for g in range(CHUNK // L):
            rvec = lane_rows + (g * L)
            cv0 = jnp.zeros((L,), jnp.int32)

            def col_step(j, carry):
                cv, acc = carry
                x = plsc.load_gather(buf, [rvec, cv])
                acc = acc + jnp.left_shift(1, jnp.left_shift(x, 3))
                return cv + 1, acc

            _, acc = lax.fori_loop(
                0, COLS, col_step, (cv0, jnp.zeros((L,), jnp.int32)),
                unroll=25,
            )

            c0 = jnp.bitwise_and(acc, 255)
            c1 = jnp.bitwise_and(lax.shift_right_logical(acc, 8), 255)
            c2 = jnp.bitwise_and(lax.shift_right_logical(acc, 16), 255)
            c3 = lax.shift_right_logical(acc, 24)
            energy = (
                c0.astype(jnp.float32) * e_splat[0]
                + c1.astype(jnp.float32) * e_splat[1]
                + c2.astype(jnp.float32) * e_splat[2]
                + c3.astype(jnp.float32) * e_splat[3]
            )
            out_v[pl.ds(c * CHUNK + g * L, L)] = energy

    pltpu.sync_copy(out_v, out_hbm.at[pl.ds(wid * RPW, RPW)])


@functools.partial(
    pl.kernel,
    out_type=jax.ShapeDtypeStruct((ROWS,), jnp.float32),
    mesh=plsc.VectorSubcoreMesh(core_axis_name="c", subcore_axis_name="s"),
    compiler_params=pltpu.CompilerParams(needs_layout_passes=False),
    scratch_types=[
        pltpu.VMEM((CHUNK, COLS), jnp.int32),
        pltpu.VMEM((L,), jnp.float32),
        pltpu.VMEM((RPW,), jnp.float32),
    ],
)
def _energy_adder(idx_hbm, es_hbm, out_hbm, buf, es_v, out_v):
    _body(idx_hbm, es_hbm, out_hbm, buf, es_v, out_v)


def kernel(element_idxs, self_energies):
    return _energy_adder(element_idxs, self_energies)


# speedup vs baseline: 1.1198x; 1.1198x over previous
import functools

import jax
import jax.numpy as jnp
from jax import lax
from jax.experimental import pallas as pl
from jax.experimental.pallas import tpu as pltpu
from jax.experimental.pallas import tpu_sc as plsc

L = 16
NC = 2
NS = 16
NW = NC * NS
ROWS = 16384
COLS = 200
RPW = ROWS // NW
GROUPS = RPW // L
CHUNK = 256


def _body(idx_hbm, es_hbm, out_hbm, buf, es_v, out_v):
    wid = lax.axis_index("s") * NC + lax.axis_index("c")

    pltpu.sync_copy(es_hbm, es_v.at[pl.ds(0, 4)])

    ev = es_v[pl.ds(0, L)]
    e_splat = [jnp.full((L,), ev[k], jnp.float32) for k in range(4)]

    lane_rows = lax.iota(jnp.int32, L)
    col_consts = [jnp.full((L,), j, jnp.int32) for j in range(COLS)]

    for c in range(RPW // CHUNK):
        pltpu.sync_copy(
            idx_hbm.at[pl.ds(wid * RPW + c * CHUNK, CHUNK), :], buf
        )

        def group_step(g, _):
            rvec = lane_rows + g * L
            acc = jnp.zeros((L,), jnp.int32)
            for j in range(COLS):
                x = plsc.load_gather(buf, [rvec, col_consts[j]])
                acc = acc + jnp.left_shift(1, jnp.left_shift(x, 3))

            c0 = jnp.bitwise_and(acc, 255)
            c1 = jnp.bitwise_and(lax.shift_right_logical(acc, 8), 255)
            c2 = jnp.bitwise_and(lax.shift_right_logical(acc, 16), 255)
            c3 = lax.shift_right_logical(acc, 24)
            energy = (
                c0.astype(jnp.float32) * e_splat[0]
                + c1.astype(jnp.float32) * e_splat[1]
                + c2.astype(jnp.float32) * e_splat[2]
                + c3.astype(jnp.float32) * e_splat[3]
            )
            out_v[pl.ds(c * CHUNK + g * L, L)] = energy
            return 0

        lax.fori_loop(0, CHUNK // L, group_step, 0)

    pltpu.sync_copy(out_v, out_hbm.at[pl.ds(wid * RPW, RPW)])


@functools.partial(
    pl.kernel,
    out_type=jax.ShapeDtypeStruct((ROWS,), jnp.float32),
    mesh=plsc.VectorSubcoreMesh(core_axis_name="c", subcore_axis_name="s"),
    compiler_params=pltpu.CompilerParams(
        needs_layout_passes=False,
        use_tc_tiling_on_sc=True,
    ),
    scratch_types=[
        pltpu.VMEM((CHUNK, COLS), jnp.int32),
        pltpu.VMEM((L,), jnp.float32),
        pltpu.VMEM((RPW,), jnp.float32),
    ],
)
def _energy_adder(idx_hbm, es_hbm, out_hbm, buf, es_v, out_v):
    _body(idx_hbm, es_hbm, out_hbm, buf, es_v, out_v)


def kernel(element_idxs, self_energies):
    return _energy_adder(element_idxs, self_energies)


# speedup vs baseline: 2.0697x; 1.8482x over previous
import functools

import jax
import jax.numpy as jnp
from jax import lax
from jax.experimental import pallas as pl
from jax.experimental.pallas import tpu as pltpu
from jax.experimental.pallas import tpu_sc as plsc

L = 16
NC = 2
NS = 16
NW = NC * NS
ROWS = 16384
COLS = 200
RPW = ROWS // NW
CHUNK = 128
NCHUNK = RPW // CHUNK
NFULL = COLS // L
TAIL0 = COLS - L


def _compute_chunk(buf, tot_v, tail_mask):
    def row_step(r, _):
        acc = jnp.zeros((L,), jnp.int32)
        for k in range(NFULL):
            x = buf[r, pl.ds(k * L, L)]
            acc = acc + jnp.left_shift(1, jnp.left_shift(x, 3))
        x = buf[r, pl.ds(TAIL0, L)]
        t = jnp.left_shift(1, jnp.left_shift(x, 3))
        acc = acc + jnp.where(tail_mask, t, 0)
        tot_v[pl.ds(r * L, L)] = plsc.cumsum(acc)
        return 0

    lax.fori_loop(0, CHUNK, row_step, 0)


def _emit_chunk(tot_v, out_v, e_splat, lane_seq, out_base):
    def group_step(g, _):
        idx = lane_seq * L + (g * (L * L) + (L - 1))
        acc = plsc.load_gather(tot_v, [idx])
        c0 = jnp.bitwise_and(acc, 255)
        c1 = jnp.bitwise_and(lax.shift_right_logical(acc, 8), 255)
        c2 = jnp.bitwise_and(lax.shift_right_logical(acc, 16), 255)
        c3 = lax.shift_right_logical(acc, 24)
        energy = (
            c0.astype(jnp.float32) * e_splat[0]
            + c1.astype(jnp.float32) * e_splat[1]
            + c2.astype(jnp.float32) * e_splat[2]
            + c3.astype(jnp.float32) * e_splat[3]
        )
        out_v[pl.ds(out_base + g * L, L)] = energy
        return 0

    lax.fori_loop(0, CHUNK // L, group_step, 0)


def _body(idx_hbm, es_hbm, out_hbm, buf0, buf1, es_v, tot_v, out_v, s0, s1):
    wid = lax.axis_index("s") * NC + lax.axis_index("c")
    row_base = wid * RPW

    pltpu.sync_copy(es_hbm, es_v.at[pl.ds(0, 4)])
    ev = es_v[pl.ds(0, L)]
    e_splat = [jnp.full((L,), ev[k], jnp.float32) for k in range(4)]
    lane_seq = lax.iota(jnp.int32, L)
    tail_mask = lane_seq >= (NFULL * L - TAIL0)

    bufs = (buf0, buf1)
    sems = (s0, s1)

    def stage(c, b):
        return pltpu.async_copy(
            idx_hbm.at[pl.ds(row_base + c * CHUNK, CHUNK), :],
            bufs[b],
            sems[b],
        )

    pending = [None, None]
    pending[0] = stage(0, 0)
    for c in range(NCHUNK):
        b = c % 2
        if c + 1 < NCHUNK:
            pending[1 - b] = stage(c + 1, 1 - b)
        pending[b].wait()
        _compute_chunk(bufs[b], tot_v, tail_mask)
        _emit_chunk(tot_v, out_v, e_splat, lane_seq, c * CHUNK)

    pltpu.sync_copy(out_v, out_hbm.at[pl.ds(wid * RPW, RPW)])


@functools.partial(
    pl.kernel,
    out_type=jax.ShapeDtypeStruct((ROWS,), jnp.float32),
    mesh=plsc.VectorSubcoreMesh(core_axis_name="c", subcore_axis_name="s"),
    compiler_params=pltpu.CompilerParams(needs_layout_passes=False),
    scratch_types=[
        pltpu.VMEM((CHUNK, COLS), jnp.int32),
        pltpu.VMEM((CHUNK, COLS), jnp.int32),
        pltpu.VMEM((L,), jnp.float32),
        pltpu.VMEM((CHUNK * L,), jnp.int32),
        pltpu.VMEM((RPW,), jnp.float32),
        pltpu.SemaphoreType.DMA,
        pltpu.SemaphoreType.DMA,
    ],
)
def _energy_adder(idx_hbm, es_hbm, out_hbm, buf0, buf1, es_v, tot_v, out_v,
                  s0, s1):
    _body(idx_hbm, es_hbm, out_hbm, buf0, buf1, es_v, tot_v, out_v, s0, s1)


def kernel(element_idxs, self_energies):
    return _energy_adder(element_idxs, self_energies)


# speedup vs baseline: 3.0268x; 1.4624x over previous
import functools

import jax
import jax.numpy as jnp
from jax import lax
from jax.experimental import pallas as pl
from jax.experimental.pallas import tpu as pltpu
from jax.experimental.pallas import tpu_sc as plsc

L = 16
NC = 2
NS = 16
NW = NC * NS
ROWS = 16384
COLS = 200
RPW = ROWS // NW
CCH = 256
NCHUNK = RPW // CCH


def _compute_chunk(buf, out_v, e_splat, out_base):
    def group_step(g, _):
        col0 = g * L
        acc = jnp.zeros((L,), jnp.int32)
        for r in range(COLS):
            x = buf[r, pl.ds(col0, L)]
            acc = acc + jnp.left_shift(1, jnp.left_shift(x, 3))

        c0 = jnp.bitwise_and(acc, 255)
        c1 = jnp.bitwise_and(lax.shift_right_logical(acc, 8), 255)
        c2 = jnp.bitwise_and(lax.shift_right_logical(acc, 16), 255)
        c3 = lax.shift_right_logical(acc, 24)
        energy = (
            c0.astype(jnp.float32) * e_splat[0]
            + c1.astype(jnp.float32) * e_splat[1]
            + c2.astype(jnp.float32) * e_splat[2]
            + c3.astype(jnp.float32) * e_splat[3]
        )
        out_v[pl.ds(out_base + g * L, L)] = energy
        return 0

    lax.fori_loop(0, CCH // L, group_step, 0)


def _body(idxT_hbm, es_hbm, out_hbm, buf0, buf1, es_v, out_v, s0, s1):
    wid = lax.axis_index("s") * NC + lax.axis_index("c")
    col_base = wid * RPW

    pltpu.sync_copy(es_hbm, es_v.at[pl.ds(0, 4)])
    ev = es_v[pl.ds(0, L)]
    e_splat = [jnp.full((L,), ev[k], jnp.float32) for k in range(4)]

    bufs = (buf0, buf1)
    sems = (s0, s1)

    def stage(c, b):
        return pltpu.async_copy(
            idxT_hbm.at[:, pl.ds(col_base + c * CCH, CCH)],
            bufs[b],
            sems[b],
        )

    pending = [None, None]
    pending[0] = stage(0, 0)
    for c in range(NCHUNK):
        b = c % 2
        if c + 1 < NCHUNK:
            pending[1 - b] = stage(c + 1, 1 - b)
        pending[b].wait()
        _compute_chunk(bufs[b], out_v, e_splat, c * CCH)

    pltpu.sync_copy(out_v, out_hbm.at[pl.ds(wid * RPW, RPW)])


@functools.partial(
    pl.kernel,
    out_type=jax.ShapeDtypeStruct((ROWS,), jnp.float32),
    mesh=plsc.VectorSubcoreMesh(core_axis_name="c", subcore_axis_name="s"),
    compiler_params=pltpu.CompilerParams(needs_layout_passes=False),
    scratch_types=[
        pltpu.VMEM((COLS, CCH), jnp.int32),
        pltpu.VMEM((COLS, CCH), jnp.int32),
        pltpu.VMEM((L,), jnp.float32),
        pltpu.VMEM((RPW,), jnp.float32),
        pltpu.SemaphoreType.DMA,
        pltpu.SemaphoreType.DMA,
    ],
)
def _energy_adder(idxT_hbm, es_hbm, out_hbm, buf0, buf1, es_v, out_v, s0, s1):
    _body(idxT_hbm, es_hbm, out_hbm, buf0, buf1, es_v, out_v, s0, s1)


def kernel(element_idxs, self_energies):
    return _energy_adder(element_idxs.T, self_energies)
